# Initial kernel scaffold; baseline (speedup 1.0000x reference)
import jax
import jax.numpy as jnp
from jax.experimental import pallas as pl


def kernel(feature, graph_node_id):
    raise NotImplementedError("write your pallas kernel here")


# speedup vs baseline: 5.7138x; 5.7138x over previous
import functools

import jax
import jax.numpy as jnp
from jax import lax
from jax.experimental import pallas as pl
from jax.experimental.pallas import tpu as pltpu
from jax.experimental.pallas import tpu_sc as plsc

_N = 100000
_D = 128
_G = 256
_L = 16
_NC = 2
_NS = 16
_V = _N // _L

_P1_MAX = -(-_V // _NS)
_P2_MAX = -(-_V // (_NC * _NS))

_mesh = plsc.VectorSubcoreMesh(
    core_axis_name="c", subcore_axis_name="s", num_cores=_NC, num_subcores=_NS
)


@functools.partial(
    pl.kernel,
    out_type=jax.ShapeDtypeStruct((_N,), jnp.float32),
    mesh=_mesh,
    compiler_params=pltpu.CompilerParams(needs_layout_passes=False),
    scratch_types=[
        pltpu.VMEM((_P1_MAX * _L,), jnp.int32),
        pltpu.VMEM((_G,), jnp.float32),
        pltpu.VMEM((_NS, _G), jnp.float32),
        pltpu.VMEM((_P2_MAX * _L,), jnp.float32),
        pltpu.VMEM_SHARED((_NS, _G), jnp.float32),
    ],
)
def _sc_node_counts(ids_hbm, cnt_hbm, ids_v, hist_v, hists_v, out_v, sh_hist):
    c = lax.axis_index("c")
    s = lax.axis_index("s")

    st1 = (s * _V) // _NS
    n1 = ((s + 1) * _V) // _NS - st1
    pltpu.sync_copy(ids_hbm.at[pl.ds(st1 * _L, _P1_MAX * _L)], ids_v)

    for j in range(_G // _L):
        hist_v[pl.ds(j * _L, _L)] = jnp.zeros((_L,), jnp.float32)

    ones = jnp.ones((_L,), jnp.float32)

    def p1_body(k, _):
        v = ids_v[pl.ds(k * _L, _L)]
        m = jnp.broadcast_to(k < n1, (_L,))
        plsc.addupdate_scatter(hist_v, [v], ones, mask=m)
        return _

    lax.fori_loop(0, _P1_MAX, p1_body, None)

    pltpu.sync_copy(hist_v, sh_hist.at[s])
    plsc.subcore_barrier()
    pltpu.sync_copy(sh_hist, hists_v)
    for j in range(_G // _L):
        acc = hists_v[0, pl.ds(j * _L, _L)]
        for t in range(1, _NS):
            acc = acc + hists_v[t, pl.ds(j * _L, _L)]
        hist_v[pl.ds(j * _L, _L)] = acc

    w = s * _NC + c
    st2 = (w * _V) // (_NC * _NS)
    pltpu.sync_copy(
        ids_hbm.at[pl.ds(st2 * _L, _P2_MAX * _L)],
        ids_v.at[pl.ds(0, _P2_MAX * _L)],
    )

    def p2_body(k, _):
        v = ids_v[pl.ds(k * _L, _L)]
        out_v[pl.ds(k * _L, _L)] = plsc.load_gather(hist_v, [v])
        return _

    lax.fori_loop(0, _P2_MAX, p2_body, None)
    pltpu.sync_copy(out_v, cnt_hbm.at[pl.ds(st2 * _L, _P2_MAX * _L)])


_B = 2000


def _tc_body(f_ref, c_ref, o_ref):
    o_ref[...] = f_ref[...] / jnp.sqrt(c_ref[...])


_tc_scale = pl.pallas_call(
    _tc_body,
    grid=(_N // _B,),
    in_specs=[
        pl.BlockSpec((_B, _D), lambda i: (i, 0)),
        pl.BlockSpec((_B, 1), lambda i: (i, 0)),
    ],
    out_specs=pl.BlockSpec((_B, _D), lambda i: (i, 0)),
    out_shape=jax.ShapeDtypeStruct((_N, _D), jnp.float32),
    compiler_params=pltpu.CompilerParams(dimension_semantics=("parallel",)),
)


def kernel(feature, graph_node_id):
    cnt = _sc_node_counts(graph_node_id)
    return _tc_scale(feature, cnt.reshape(_N, 1))


# speedup vs baseline: 7.9522x; 1.3918x over previous
import functools

import jax
import jax.numpy as jnp
from jax import lax
from jax.experimental import pallas as pl
from jax.experimental.pallas import tpu as pltpu
from jax.experimental.pallas import tpu_sc as plsc

_N = 100000
_D = 128
_G = 256
_L = 16
_NC = 2
_NS = 16
_V = _N // _L

_P1_MAX = -(-_V // _NS)

_mesh = plsc.VectorSubcoreMesh(
    core_axis_name="c", subcore_axis_name="s", num_cores=_NC, num_subcores=_NS
)


@functools.partial(
    pl.kernel,
    out_type=jax.ShapeDtypeStruct((_NS, _D), jnp.float32),
    mesh=_mesh,
    compiler_params=pltpu.CompilerParams(needs_layout_passes=False),
    scratch_types=[
        pltpu.VMEM((_P1_MAX * _L,), jnp.int32),
        pltpu.VMEM((_G,), jnp.float32),
        pltpu.VMEM((_NS, _G), jnp.float32),
        pltpu.VMEM_SHARED((_NS, _G), jnp.float32),
    ],
)
def _sc_graph_counts(ids_hbm, cnt_hbm, ids_v, hist_v, hists_v, sh_hist):
    c = lax.axis_index("c")
    s = lax.axis_index("s")

    st1 = (s * _V) // _NS
    n1 = ((s + 1) * _V) // _NS - st1
    pltpu.sync_copy(ids_hbm.at[pl.ds(st1 * _L, _P1_MAX * _L)], ids_v)

    for j in range(_G // _L):
        hist_v[pl.ds(j * _L, _L)] = jnp.zeros((_L,), jnp.float32)

    ones = jnp.ones((_L,), jnp.float32)

    def p1_body(k, _):
        v = ids_v[pl.ds(k * _L, _L)]
        plsc.addupdate_scatter(hist_v, [v], ones)
        return _

    lax.fori_loop(0, n1, p1_body, None)

    pltpu.sync_copy(hist_v, sh_hist.at[s])
    plsc.subcore_barrier()
    pltpu.sync_copy(sh_hist, hists_v)
    for j in range(_G // _L):
        acc = hists_v[0, pl.ds(j * _L, _L)]
        for t in range(1, _NS):
            acc = acc + hists_v[t, pl.ds(j * _L, _L)]
        hist_v[pl.ds(j * _L, _L)] = acc

    @pl.when(c == 0)
    def _():
        pltpu.sync_copy(
            hist_v.at[pl.ds(s * _L, _L)], cnt_hbm.at[s, pl.ds(0, _L)]
        )


_B = 2000


def _tc_body(f_ref, i_ref, c_ref, o_ref):
    c2 = c_ref[...][:, :_L]
    inv2 = 1.0 / jnp.sqrt(jnp.maximum(c2, 1.0))

    ids = i_ref[0]
    hi = jnp.broadcast_to(ids >> 4, (_L, _B))
    lo = jnp.broadcast_to(ids & 15, (_L, _B))
    rows = lax.broadcasted_iota(jnp.int32, (_L, _B), 0)
    oha = jnp.where(rows == hi, 1.0, 0.0)
    ohb = jnp.where(rows == lo, 1.0, 0.0)

    m = lax.dot_general(inv2, oha, (((0,), (0,)), ((), ())))
    scale = lax.dot_general(
        ohb * m, jnp.ones((_L, 1), jnp.float32), (((0,), (0,)), ((), ()))
    )
    o_ref[...] = f_ref[...] * scale


_tc_scale = pl.pallas_call(
    _tc_body,
    grid=(_N // _B,),
    in_specs=[
        pl.BlockSpec((_B, _D), lambda i: (i, 0)),
        pl.BlockSpec((1, 1, _B), lambda i: (i, 0, 0)),
        pl.BlockSpec((_NS, _D), lambda i: (0, 0)),
    ],
    out_specs=pl.BlockSpec((_B, _D), lambda i: (i, 0)),
    out_shape=jax.ShapeDtypeStruct((_N, _D), jnp.float32),
    compiler_params=pltpu.CompilerParams(dimension_semantics=("parallel",)),
)


def kernel(feature, graph_node_id):
    cnt = _sc_graph_counts(graph_node_id)
    ids3 = graph_node_id.reshape(_N // _B, 1, _B)
    return _tc_scale(feature, ids3, cnt)


# speedup vs baseline: 8.3759x; 1.0533x over previous
import functools

import jax
import jax.numpy as jnp
from jax import lax
from jax.experimental import pallas as pl
from jax.experimental.pallas import tpu as pltpu
from jax.experimental.pallas import tpu_sc as plsc

_N = 100000
_D = 128
_G = 256
_L = 16
_NC = 2
_NS = 16
_V = _N // _L

_P1_MAX = -(-_V // _NS)

_mesh = plsc.VectorSubcoreMesh(
    core_axis_name="c", subcore_axis_name="s", num_cores=_NC, num_subcores=_NS
)


@functools.partial(
    pl.kernel,
    out_type=jax.ShapeDtypeStruct((_NS, _D), jnp.float32),
    mesh=_mesh,
    compiler_params=pltpu.CompilerParams(needs_layout_passes=False),
    scratch_types=[
        pltpu.VMEM((_P1_MAX * _L,), jnp.int32),
        pltpu.VMEM((_G,), jnp.float32),
        pltpu.VMEM((_NS, _G), jnp.float32),
        pltpu.VMEM_SHARED((_NS, _G), jnp.float32),
    ],
)
def _sc_graph_counts(ids_hbm, cnt_hbm, ids_v, hist_v, hists_v, sh_hist):
    c = lax.axis_index("c")
    s = lax.axis_index("s")

    st1 = (s * _V) // _NS
    n1 = ((s + 1) * _V) // _NS - st1
    pltpu.sync_copy(ids_hbm.at[pl.ds(st1 * _L, _P1_MAX * _L)], ids_v)

    for j in range(_G // _L):
        hist_v[pl.ds(j * _L, _L)] = jnp.zeros((_L,), jnp.float32)

    ones = jnp.ones((_L,), jnp.float32)

    def p1_body(k, _):
        v = ids_v[pl.ds(k * _L, _L)]
        plsc.addupdate_scatter(hist_v, [v], ones)
        return _

    lax.fori_loop(0, n1, p1_body, None)

    pltpu.sync_copy(hist_v, sh_hist.at[s])
    plsc.subcore_barrier()
    pltpu.sync_copy(sh_hist, hists_v)
    for j in range(_G // _L):
        acc = hists_v[0, pl.ds(j * _L, _L)]
        for t in range(1, _NS):
            acc = acc + hists_v[t, pl.ds(j * _L, _L)]
        hist_v[pl.ds(j * _L, _L)] = acc

    @pl.when(c == 0)
    def _():
        pltpu.sync_copy(
            hist_v.at[pl.ds(s * _L, _L)], cnt_hbm.at[s, pl.ds(0, _L)]
        )


_B = 2048
_NB = -(-_N // _B)


def _tc_body(f_ref, i_ref, c_ref, o_ref):
    c2 = c_ref[...][:, :_L]
    inv2 = 1.0 / jnp.sqrt(jnp.maximum(c2, 1.0))

    ids = i_ref[...]
    hi = jnp.broadcast_to(ids >> 4, (_L, _B))
    lo = jnp.broadcast_to(ids & 15, (_L, _B))
    rows = lax.broadcasted_iota(jnp.int32, (_L, _B), 0)
    oha = jnp.where(rows == hi, 1.0, 0.0)
    ohb = jnp.where(rows == lo, 1.0, 0.0)

    m = lax.dot_general(inv2, oha, (((0,), (0,)), ((), ())))
    scale = lax.dot_general(
        ohb * m, jnp.ones((_L, _D), jnp.float32), (((0,), (0,)), ((), ()))
    )
    o_ref[...] = f_ref[...] * scale


_tc_scale = pl.pallas_call(
    _tc_body,
    grid=(_NB,),
    in_specs=[
        pl.BlockSpec((_B, _D), lambda i: (i, 0)),
        pl.BlockSpec((_B,), lambda i: (i,)),
        pl.BlockSpec((_NS, _D), lambda i: (0, 0)),
    ],
    out_specs=pl.BlockSpec((_B, _D), lambda i: (i, 0)),
    out_shape=jax.ShapeDtypeStruct((_N, _D), jnp.float32),
    compiler_params=pltpu.CompilerParams(dimension_semantics=("parallel",)),
)


def kernel(feature, graph_node_id):
    cnt = _sc_graph_counts(graph_node_id)
    return _tc_scale(feature, graph_node_id, cnt)


# speedup vs baseline: 9.0322x; 1.0784x over previous
import functools

import jax
import jax.numpy as jnp
from jax import lax
from jax.experimental import pallas as pl
from jax.experimental.pallas import tpu as pltpu
from jax.experimental.pallas import tpu_sc as plsc

_N = 100000
_D = 128
_G = 256
_L = 16
_NC = 2
_NS = 16
_V = _N // _L

_NW = _NC * _NS
_P1_MAX = -(-_V // _NW)
_STR = _G + 1
_HSZ = _STR * _L

_mesh = plsc.VectorSubcoreMesh(
    core_axis_name="c", subcore_axis_name="s", num_cores=_NC, num_subcores=_NS
)


@functools.partial(
    pl.kernel,
    out_type=jax.ShapeDtypeStruct((_NC, _NS, _D), jnp.float32),
    mesh=_mesh,
    compiler_params=pltpu.CompilerParams(needs_layout_passes=False),
    scratch_types=[
        pltpu.VMEM((_P1_MAX * _L,), jnp.int32),
        pltpu.VMEM((_HSZ,), jnp.float32),
        pltpu.VMEM((_G,), jnp.float32),
        pltpu.VMEM((_NS, _G), jnp.float32),
        pltpu.VMEM_SHARED((_NS, _G), jnp.float32),
    ],
)
def _sc_graph_counts(ids_hbm, cnt_hbm, ids_v, sub_v, hist_v, hists_v, sh_hist):
    c = lax.axis_index("c")
    s = lax.axis_index("s")
    w = s * _NC + c

    st1 = (w * _V) // _NW
    n1 = ((w + 1) * _V) // _NW - st1
    pltpu.sync_copy(ids_hbm.at[pl.ds(st1 * _L, _P1_MAX * _L)], ids_v)

    for j in range(_HSZ // _L):
        sub_v[pl.ds(j * _L, _L)] = jnp.zeros((_L,), jnp.float32)

    ones = jnp.ones((_L,), jnp.float32)
    offs = lax.iota(jnp.int32, _L) * _STR

    def p1_body(k, _):
        v = ids_v[pl.ds(k * _L, _L)]
        plsc.addupdate_scatter(sub_v, [v + offs], ones)
        return _

    lax.fori_loop(0, n1, p1_body, None)

    for j in range(_G // _L):
        acc = sub_v[pl.ds(j * _L, _L)]
        for t in range(1, _L):
            acc = acc + sub_v[pl.ds(t * _STR + j * _L, _L)]
        hist_v[pl.ds(j * _L, _L)] = acc

    pltpu.sync_copy(hist_v, sh_hist.at[s])
    plsc.subcore_barrier()
    pltpu.sync_copy(sh_hist, hists_v)
    for j in range(_G // _L):
        acc = hists_v[0, pl.ds(j * _L, _L)]
        for t in range(1, _NS):
            acc = acc + hists_v[t, pl.ds(j * _L, _L)]
        hist_v[pl.ds(j * _L, _L)] = acc

    pltpu.sync_copy(
        hist_v.at[pl.ds(s * _L, _L)], cnt_hbm.at[c, s, pl.ds(0, _L)]
    )


_B = 2048
_NB = -(-_N // _B)


def _tc_body(f_ref, i_ref, c_ref, o_ref):
    c2 = (c_ref[0] + c_ref[1])[:, :_L]
    inv2 = 1.0 / jnp.sqrt(jnp.maximum(c2, 1.0))

    ids = i_ref[...]
    hi = jnp.broadcast_to(ids >> 4, (_L, _B))
    lo = jnp.broadcast_to(ids & 15, (_L, _B))
    rows = lax.broadcasted_iota(jnp.int32, (_L, _B), 0)
    oha = jnp.where(rows == hi, 1.0, 0.0)
    ohb = jnp.where(rows == lo, 1.0, 0.0)

    m = lax.dot_general(inv2, oha, (((0,), (0,)), ((), ())))
    scale = lax.dot_general(
        ohb * m, jnp.ones((_L, _D), jnp.float32), (((0,), (0,)), ((), ()))
    )
    o_ref[...] = f_ref[...] * scale


_tc_scale = pl.pallas_call(
    _tc_body,
    grid=(_NB,),
    in_specs=[
        pl.BlockSpec((_B, _D), lambda i: (i, 0)),
        pl.BlockSpec((_B,), lambda i: (i,)),
        pl.BlockSpec((_NC, _NS, _D), lambda i: (0, 0, 0)),
    ],
    out_specs=pl.BlockSpec((_B, _D), lambda i: (i, 0)),
    out_shape=jax.ShapeDtypeStruct((_N, _D), jnp.float32),
    compiler_params=pltpu.CompilerParams(dimension_semantics=("parallel",)),
)


def kernel(feature, graph_node_id):
    cnt = _sc_graph_counts(graph_node_id)
    return _tc_scale(feature, graph_node_id, cnt)


# speedup vs baseline: 11.2055x; 1.2406x over previous
import functools

import jax
import jax.numpy as jnp
from jax import lax
from jax.experimental import pallas as pl
from jax.experimental.pallas import tpu as pltpu
from jax.experimental.pallas import tpu_sc as plsc

_N = 100000
_D = 128
_G = 256
_L = 16
_NC = 2
_NS = 16
_V = _N // _L

_NW = _NC * _NS
_P1_MAX = -(-_V // _NW)
_STR = _G + 1
_HSZ = _STR * _L

_mesh = plsc.VectorSubcoreMesh(
    core_axis_name="c", subcore_axis_name="s", num_cores=_NC, num_subcores=_NS
)


@functools.partial(
    pl.kernel,
    out_type=jax.ShapeDtypeStruct((_NC, _NS, _D), jnp.float32),
    mesh=_mesh,
    compiler_params=pltpu.CompilerParams(needs_layout_passes=False),
    scratch_types=[
        pltpu.VMEM((_P1_MAX * _L,), jnp.int32),
        pltpu.VMEM((_HSZ,), jnp.float32),
        pltpu.VMEM((_G,), jnp.float32),
        pltpu.VMEM((_NS, _G), jnp.float32),
        pltpu.VMEM_SHARED((_NS, _G), jnp.float32),
    ],
)
def _sc_graph_counts(ids_hbm, cnt_hbm, ids_v, sub_v, hist_v, hists_v, sh_hist):
    c = lax.axis_index("c")
    s = lax.axis_index("s")
    w = s * _NC + c

    st1 = (w * _V) // _NW
    n1 = ((w + 1) * _V) // _NW - st1
    pltpu.sync_copy(ids_hbm.at[pl.ds(st1 * _L, _P1_MAX * _L)], ids_v)

    for j in range(_HSZ // _L):
        sub_v[pl.ds(j * _L, _L)] = jnp.zeros((_L,), jnp.float32)

    ones = jnp.ones((_L,), jnp.float32)
    offs = lax.iota(jnp.int32, _L) * _STR

    def p1_body(k, _):
        v = ids_v[pl.ds(k * _L, _L)]
        plsc.addupdate_scatter(sub_v, [v + offs], ones)
        return _

    lax.fori_loop(0, n1, p1_body, None)

    for j in range(_G // _L):
        acc = sub_v[pl.ds(j * _L, _L)]
        for t in range(1, _L):
            acc = acc + sub_v[pl.ds(t * _STR + j * _L, _L)]
        hist_v[pl.ds(j * _L, _L)] = acc

    pltpu.sync_copy(hist_v, sh_hist.at[s])
    plsc.subcore_barrier()
    pltpu.sync_copy(sh_hist, hists_v)
    for j in range(_G // _L):
        acc = hists_v[0, pl.ds(j * _L, _L)]
        for t in range(1, _NS):
            acc = acc + hists_v[t, pl.ds(j * _L, _L)]
        hist_v[pl.ds(j * _L, _L)] = acc

    pltpu.sync_copy(
        hist_v.at[pl.ds(s * _L, _L)], cnt_hbm.at[c, s, pl.ds(0, _L)]
    )


_B = 4096
_NB = -(-_N // _B)


def _tc_body(f_ref, i_ref, c_ref, o_ref):
    c2 = (c_ref[0] + c_ref[1])[:, :_L]
    inv2 = 1.0 / jnp.sqrt(jnp.maximum(c2, 1.0))

    ids = i_ref[...]
    hi = jnp.broadcast_to(ids >> 4, (_L, _B))
    lo = jnp.broadcast_to(ids & 15, (_L, _B))
    rows = lax.broadcasted_iota(jnp.int32, (_L, _B), 0)
    oha = jnp.where(rows == hi, 1.0, 0.0)
    ohb = jnp.where(rows == lo, 1.0, 0.0)

    m = lax.dot_general(inv2, oha, (((0,), (0,)), ((), ())))
    scale = lax.dot_general(
        ohb * m, jnp.ones((_L, _D), jnp.float32), (((0,), (0,)), ((), ()))
    )
    o_ref[...] = f_ref[...] * scale


_tc_scale = pl.pallas_call(
    _tc_body,
    grid=(_NB,),
    in_specs=[
        pl.BlockSpec((_B, _D), lambda i: (i, 0)),
        pl.BlockSpec((_B,), lambda i: (i,)),
        pl.BlockSpec((_NC, _NS, _D), lambda i: (0, 0, 0)),
    ],
    out_specs=pl.BlockSpec((_B, _D), lambda i: (i, 0)),
    out_shape=jax.ShapeDtypeStruct((_N, _D), jnp.float32),
    compiler_params=pltpu.CompilerParams(dimension_semantics=("parallel",)),
)


def kernel(feature, graph_node_id):
    cnt = _sc_graph_counts(graph_node_id)
    return _tc_scale(feature, graph_node_id, cnt)
